# baseline (speedup 1.0000x reference)
import jax
import jax.numpy as jnp
from jax.experimental import pallas as pl
from jax.experimental.pallas import tpu as pltpu

BSZ = 8
WIN = 100
NN = 512
IN_CH = 64
HID = 16
HEADS = 4
OUT_CH = 64

_CN = (((1,), (1,)), ((), ()))
_C0 = (((0,), (1,)), ((), ()))


def _dot_t(a, b):
    return jax.lax.dot_general(a, b, _CN, preferred_element_type=jnp.float32)


def _attend(x0, xl0, get_cnt, wr, br, att_ref, att_v, heads, ch):
    xr = _dot_t(x0, wr) + br
    xlT = xl0.T
    ss = []
    for h in range(heads):
        sl = slice(h * ch, (h + 1) * ch)
        ar = _dot_t(xr[:, sl], att_v[:, sl]) * 0.6
        al = _dot_t(att_v[:, sl], xl0[:, sl]) * 0.6
        s = ar + al
        for c in range(ch):
            k = h * ch + c
            z = xr[:, k:k + 1] + xlT[k:k + 1, :]
            s = s + (0.4 * att_ref[h, c]) * jnp.abs(z)
        ss.append(s)
    cnt = get_cnt()
    outs = []
    for h in range(heads):
        sl = slice(h * ch, (h + 1) * ch)
        s = ss[h]
        amax = jnp.max(s, axis=1, keepdims=True)
        p = cnt * jnp.exp(s - amax)
        den = jnp.sum(p, axis=1, keepdims=True) + 1e-16
        o = jnp.dot(p, xl0[:, sl],
                    preferred_element_type=jnp.float32) / den
        outs.append(o)
    return outs[0] if heads == 1 else jnp.concatenate(outs, axis=1)


def _enc_kernel(att1_ref, att2_ref, inp0_ref, cnt_hbm, rest_hbm,
                wemb_ref, bemb_ref,
                wl1_ref, bl1_ref, wr1_ref, br1_ref, bias1_ref,
                wl2_ref, bl2_ref, wr2_ref, br2_ref, bias2_ref,
                att1v_ref, att2v_ref, out_ref, cnt_v, rest_v, sem_c, sem_r):
    c_cp = pltpu.make_async_copy(cnt_hbm, cnt_v, sem_c)
    c_cp.start()
    r_cp = pltpu.make_async_copy(rest_hbm, rest_v, sem_r)
    r_cp.start()

    wemb = wemb_ref[...]
    x0 = (jax.lax.dot_general(inp0_ref[...], wemb, _C0,
                              preferred_element_type=jnp.float32)
          + bemb_ref[...])

    xl1_0 = _dot_t(x0, wl1_ref[...]) + bl1_ref[...]

    def cnt_first():
        c_cp.wait()
        return cnt_v[...]

    att_out1 = _attend(x0, xl1_0, cnt_first, wr1_ref[...], br1_ref[...],
                       att1_ref, att1v_ref[...], HEADS, HID)
    cnt = cnt_v[...]

    r_cp.wait()
    xs = [jax.lax.dot_general(rest_v[b], wemb, _C0,
                              preferred_element_type=jnp.float32)
          for b in range(BSZ - 1)]
    x_rest = jnp.concatenate(xs, axis=0) + bemb_ref[...]
    xl1_tail = _dot_t(x_rest, wl1_ref[...]) + bl1_ref[...]
    h1 = jnp.concatenate([att_out1, xl1_tail], axis=0) + bias1_ref[...]
    x2 = jnp.where(h1 > 0, h1, jnp.exp(h1) - 1.0)

    xl2 = _dot_t(x2, wl2_ref[...]) + bl2_ref[...]
    att_out2 = _attend(x2[:NN], xl2[:NN], lambda: cnt, wr2_ref[...],
                       br2_ref[...], att2_ref, att2v_ref[...], 1, OUT_CH)
    h2 = jnp.concatenate([att_out2, xl2[NN:]], axis=0) + bias2_ref[...]

    out_ref[...] = jnp.mean(h2.reshape(BSZ, NN, OUT_CH), axis=1)


def kernel(input, adj_mtx, W_emb, b_emb, Wl1, bl1, Wr1, br1, att1, bias1,
           Wl2, bl2, Wr2, br2, att2, bias2):
    cnt = (BSZ * (adj_mtx != 0).astype(jnp.float32).T
           + jnp.eye(NN, dtype=jnp.float32))

    smem = pl.BlockSpec(memory_space=pltpu.SMEM)
    hbm = pl.BlockSpec(memory_space=pl.ANY)
    row = lambda v: v.reshape(1, -1)

    return pl.pallas_call(
        _enc_kernel,
        in_specs=([smem, smem, pl.BlockSpec(), hbm, hbm]
                  + [pl.BlockSpec()] * 14),
        out_specs=pl.BlockSpec(),
        out_shape=jax.ShapeDtypeStruct((BSZ, OUT_CH), jnp.float32),
        scratch_shapes=[
            pltpu.VMEM((NN, NN), jnp.float32),
            pltpu.VMEM((BSZ - 1, WIN, NN), jnp.float32),
            pltpu.SemaphoreType.DMA,
            pltpu.SemaphoreType.DMA,
        ],
    )(att1, att2, input[0], cnt, input[1:], W_emb, row(b_emb),
      Wl1, row(bl1), Wr1, row(br1), row(bias1),
      Wl2, row(bl2), Wr2, row(br2), row(bias2),
      att1.reshape(1, HEADS * HID), att2.reshape(1, OUT_CH))


# speedup vs baseline: 1.0297x; 1.0297x over previous
import jax
import jax.numpy as jnp
from jax.experimental import pallas as pl
from jax.experimental.pallas import tpu as pltpu

BSZ = 8
WIN = 100
NN = 512
IN_CH = 64
HID = 16
HEADS = 4
OUT_CH = 64

_CN = (((1,), (1,)), ((), ()))


def _dot_t(a, b):
    return jax.lax.dot_general(a, b, _CN, preferred_element_type=jnp.float32)


def _attend(x0, xl0, cnt, wr, br, att_ref, att_v, heads, ch):
    xr = _dot_t(x0, wr) + br
    xlT = xl0.T
    outs = []
    for h in range(heads):
        sl = slice(h * ch, (h + 1) * ch)
        ar = _dot_t(xr[:, sl], att_v[:, sl]) * 0.6
        al = _dot_t(att_v[:, sl], xl0[:, sl]) * 0.6
        s = ar + al
        for c in range(ch):
            k = h * ch + c
            z = xr[:, k:k + 1] + xlT[k:k + 1, :]
            s = s + (0.4 * att_ref[h, c]) * jnp.abs(z)
        amax = jnp.max(s, axis=1, keepdims=True)
        p = cnt * jnp.exp(s - amax)
        den = jnp.sum(p, axis=1, keepdims=True) + 1e-16
        o = jnp.dot(p, xl0[:, h * ch:(h + 1) * ch],
                    preferred_element_type=jnp.float32) / den
        outs.append(o)
    return outs[0] if heads == 1 else jnp.concatenate(outs, axis=1)


def _enc_kernel(att1_ref, att2_ref, xt_ref, cnt_ref, wemb_ref, bemb_ref,
                wl1_ref, bl1_ref, wr1_ref, br1_ref, bias1_ref,
                wl2_ref, bl2_ref, wr2_ref, br2_ref, bias2_ref,
                att1v_ref, att2v_ref, out_ref):
    cnt = cnt_ref[...]
    xs = [jax.lax.dot_general(xt_ref[b], wemb_ref[...],
                              (((0,), (1,)), ((), ())),
                              preferred_element_type=jnp.float32)
          for b in range(BSZ)]
    x = jnp.concatenate(xs, axis=0) + bemb_ref[...]

    xl1 = _dot_t(x, wl1_ref[...]) + bl1_ref[...]
    att_out1 = _attend(x[:NN], xl1[:NN], cnt, wr1_ref[...], br1_ref[...],
                       att1_ref, att1v_ref[...], HEADS, HID)
    h1 = jnp.concatenate([att_out1, xl1[NN:]], axis=0) + bias1_ref[...]
    x2 = jnp.where(h1 > 0, h1, jnp.exp(h1) - 1.0)

    xl2 = _dot_t(x2, wl2_ref[...]) + bl2_ref[...]
    att_out2 = _attend(x2[:NN], xl2[:NN], cnt, wr2_ref[...], br2_ref[...],
                       att2_ref, att2v_ref[...], 1, OUT_CH)
    h2 = jnp.concatenate([att_out2, xl2[NN:]], axis=0) + bias2_ref[...]

    out_ref[...] = jnp.mean(h2.reshape(BSZ, NN, OUT_CH), axis=1)


def kernel(input, adj_mtx, W_emb, b_emb, Wl1, bl1, Wr1, br1, att1, bias1,
           Wl2, bl2, Wr2, br2, att2, bias2):
    cnt = (BSZ * (adj_mtx != 0).astype(jnp.float32).T
           + jnp.eye(NN, dtype=jnp.float32))

    smem = pl.BlockSpec(memory_space=pltpu.SMEM)
    row = lambda v: v.reshape(1, -1)

    return pl.pallas_call(
        _enc_kernel,
        in_specs=[smem, smem] + [pl.BlockSpec()] * 16,
        out_specs=pl.BlockSpec(),
        out_shape=jax.ShapeDtypeStruct((BSZ, OUT_CH), jnp.float32),
    )(att1, att2, input, cnt, W_emb, row(b_emb),
      Wl1, row(bl1), Wr1, row(br1), row(bias1),
      Wl2, row(bl2), Wr2, row(br2), row(bias2),
      att1.reshape(1, HEADS * HID), att2.reshape(1, OUT_CH))
